# baseline (speedup 1.0000x reference)
import functools

import jax
import jax.numpy as jnp
from jax import lax
from jax.experimental import pallas as pl
from jax.experimental.pallas import tpu as pltpu
from jax.experimental.pallas import tpu_sc as plsc

N = 10000
E = 320000
D = 128
DA = 144

NC = 2
NS = 16
NW = NC * NS

CH = 64
NCH = 160
E_PAD = NW * NCH * CH
NIDX = NCH + 2
ROWS_PER_TILE = N // NS

BLK = 2000


def _sc_body(xaug_hbm, src_hbm, dst_hbm, zeros_hbm, out_hbm,
             src_v, dst_v, rows0, rows1, acc, sem0, sem1):
  c = lax.axis_index("c")
  s = lax.axis_index("s")
  wid = c * NS + s
  pltpu.sync_copy(zeros_hbm, acc.at[pl.ds(s * ROWS_PER_TILE, ROWS_PER_TILE)])
  pltpu.sync_copy(src_hbm.at[pl.ds(wid * NIDX, NIDX)], src_v)
  pltpu.sync_copy(dst_hbm.at[pl.ds(wid * NIDX, NIDX)], dst_v)
  plsc.subcore_barrier()

  pltpu.async_copy(xaug_hbm.at[src_v.at[0]], rows0, sem0)
  pltpu.async_copy(xaug_hbm.at[src_v.at[1]], rows1, sem1)

  def body(jj, carry):
    j = jj * 2
    pltpu.make_async_copy(xaug_hbm.at[src_v.at[j]], rows0, sem0).wait()
    pltpu.sync_copy(rows0, acc.at[dst_v.at[j]], add=True)
    pltpu.async_copy(xaug_hbm.at[src_v.at[j + 2]], rows0, sem0)
    pltpu.make_async_copy(xaug_hbm.at[src_v.at[j + 1]], rows1, sem1).wait()
    pltpu.sync_copy(rows1, acc.at[dst_v.at[j + 1]], add=True)
    pltpu.async_copy(xaug_hbm.at[src_v.at[j + 3]], rows1, sem1)
    return carry

  lax.fori_loop(0, NCH // 2, body, 0)
  pltpu.make_async_copy(xaug_hbm.at[src_v.at[NCH]], rows0, sem0).wait()
  pltpu.make_async_copy(xaug_hbm.at[src_v.at[NCH + 1]], rows1, sem1).wait()
  plsc.subcore_barrier()
  pltpu.sync_copy(acc.at[pl.ds(s * ROWS_PER_TILE, ROWS_PER_TILE)],
                  out_hbm.at[c, pl.ds(s * ROWS_PER_TILE, ROWS_PER_TILE)])


_sc_gather_scatter = functools.partial(
    pl.kernel,
    out_type=jax.ShapeDtypeStruct((NC, N, DA), jnp.float32),
    mesh=plsc.VectorSubcoreMesh(
        core_axis_name="c", subcore_axis_name="s", num_cores=NC,
        num_subcores=NS),
    scratch_types=[
        pltpu.VMEM((NIDX, CH), jnp.int32),
        pltpu.VMEM((NIDX, CH), jnp.int32),
        pltpu.VMEM((CH, DA), jnp.float32),
        pltpu.VMEM((CH, DA), jnp.float32),
        pltpu.VMEM_SHARED((N, DA), jnp.float32),
        pltpu.SemaphoreType.DMA,
        pltpu.SemaphoreType.DMA,
    ],
    compiler_params=pltpu.CompilerParams(use_tc_tiling_on_sc=False),
)(_sc_body)


def _tc_body(p_ref, x_ref, wm_ref, ws_ref, b_ref, o_ref):
  p = p_ref[0] + p_ref[1]
  deg = jnp.maximum(p[:, D:D + 1], 1.0)
  agg = p[:, :D] / deg
  h = jnp.dot(agg, wm_ref[...], preferred_element_type=jnp.float32)
  h = h + jnp.dot(x_ref[...], ws_ref[...], preferred_element_type=jnp.float32)
  h = h + b_ref[...]
  o_ref[...] = jnp.maximum(h, 0.0)


def kernel(x, edge_index, W_msg, W_self, b):
  xaug = jnp.zeros((N + 8, DA), jnp.float32)
  xaug = xaug.at[:N, :D].set(x).at[:N, D].set(1.0)
  src = edge_index[0].astype(jnp.int32)
  dst = edge_index[1].astype(jnp.int32)
  pad = E_PAD - E
  src_p = jnp.concatenate([src, jnp.full((pad,), N, jnp.int32)])
  src_p = src_p.reshape(NW, NCH, CH)
  src_p = jnp.concatenate(
      [src_p, jnp.full((NW, 2, CH), N, jnp.int32)], axis=1)
  src_p = src_p.reshape(NW * NIDX, CH)
  dst_p = jnp.concatenate([dst, jnp.zeros((pad,), jnp.int32)])
  dst_p = dst_p.reshape(NW, NCH, CH)
  dst_p = jnp.concatenate(
      [dst_p, jnp.zeros((NW, 2, CH), jnp.int32)], axis=1)
  dst_p = dst_p.reshape(NW * NIDX, CH)
  zeros_blk = jnp.zeros((ROWS_PER_TILE, DA), jnp.float32)

  partial = _sc_gather_scatter(xaug, src_p, dst_p, zeros_blk)

  out = pl.pallas_call(
      _tc_body,
      grid=(N // BLK,),
      in_specs=[
          pl.BlockSpec((NC, BLK, DA), lambda i: (0, i, 0)),
          pl.BlockSpec((BLK, D), lambda i: (i, 0)),
          pl.BlockSpec((D, D), lambda i: (0, 0)),
          pl.BlockSpec((D, D), lambda i: (0, 0)),
          pl.BlockSpec((1, D), lambda i: (0, 0)),
      ],
      out_specs=pl.BlockSpec((BLK, D), lambda i: (i, 0)),
      out_shape=jax.ShapeDtypeStruct((N, D), jnp.float32),
  )(partial, x, W_msg, W_self, b.reshape(1, D))
  return out


# speedup vs baseline: 2.4877x; 2.4877x over previous
import functools

import jax
import jax.numpy as jnp
from jax import lax
from jax.experimental import pallas as pl
from jax.experimental.pallas import tpu as pltpu
from jax.experimental.pallas import tpu_sc as plsc

N = 10000
E = 320000
D = 128
DA = 144

NC = 2
NS = 16
NW = NC * NS

CH = 96
NCH = 108
E_PAD = NW * NCH * CH
NIDX = NCH + 2
ROWS_PER_TILE = N // NS

BLK = 2000


def _sc_body(xaug_hbm, enc_hbm, zeros_hbm, out_hbm,
             pk_v, srcA, dstA, srcB, dstB, rowsA, rowsB, acc,
             semi, gsA, gsB, ssA, ssB):
  c = lax.axis_index("c")
  s = lax.axis_index("s")
  wid = c * NS + s
  icp = pltpu.async_copy(enc_hbm.at[pl.ds(wid * NIDX, NIDX)], pk_v, semi)
  pltpu.sync_copy(zeros_hbm, rowsA)
  base = s * ROWS_PER_TILE
  nfull = ROWS_PER_TILE // CH
  rem = ROWS_PER_TILE - nfull * CH
  for k in range(nfull):
    pltpu.sync_copy(rowsA, acc.at[pl.ds(base + k * CH, CH)])
  pltpu.sync_copy(rowsA.at[pl.ds(0, rem)],
                  acc.at[pl.ds(base + nfull * CH, rem)])
  icp.wait()
  plsc.subcore_barrier()

  def unpack(j, src_b, dst_b):
    for k in range(CH // 16):
      v = pk_v[j, pl.ds(k * 16, 16)]
      src_b[pl.ds(k * 16, 16)] = v & 0xFFFF
      dst_b[pl.ds(k * 16, 16)] = v >> 16

  unpack(0, srcA, dstA)
  pltpu.async_copy(xaug_hbm.at[srcA], rowsA, gsA)
  unpack(1, srcB, dstB)
  pltpu.async_copy(xaug_hbm.at[srcB], rowsB, gsB)

  def body(jj, carry):
    j = jj * 2
    pltpu.make_async_copy(xaug_hbm.at[srcA], rowsA, gsA).wait()
    pltpu.async_copy(rowsA, acc.at[dstA], ssA, add=True)
    pltpu.make_async_copy(xaug_hbm.at[srcB], rowsB, gsB).wait()
    pltpu.async_copy(rowsB, acc.at[dstB], ssB, add=True)
    pltpu.make_async_copy(rowsA, acc.at[dstA], ssA).wait()
    unpack(j + 2, srcA, dstA)
    pltpu.async_copy(xaug_hbm.at[srcA], rowsA, gsA)
    pltpu.make_async_copy(rowsB, acc.at[dstB], ssB).wait()
    unpack(j + 3, srcB, dstB)
    pltpu.async_copy(xaug_hbm.at[srcB], rowsB, gsB)
    return carry

  lax.fori_loop(0, NCH // 2, body, 0)
  pltpu.make_async_copy(xaug_hbm.at[srcA], rowsA, gsA).wait()
  pltpu.make_async_copy(xaug_hbm.at[srcB], rowsB, gsB).wait()
  plsc.subcore_barrier()
  pltpu.sync_copy(acc.at[pl.ds(base, ROWS_PER_TILE)],
                  out_hbm.at[c, pl.ds(base, ROWS_PER_TILE)])


_sc_gather_scatter = functools.partial(
    pl.kernel,
    out_type=jax.ShapeDtypeStruct((NC, N, DA), jnp.float32),
    mesh=plsc.VectorSubcoreMesh(
        core_axis_name="c", subcore_axis_name="s", num_cores=NC,
        num_subcores=NS),
    scratch_types=[
        pltpu.VMEM((NIDX, CH), jnp.int32),
        pltpu.VMEM((CH,), jnp.int32),
        pltpu.VMEM((CH,), jnp.int32),
        pltpu.VMEM((CH,), jnp.int32),
        pltpu.VMEM((CH,), jnp.int32),
        pltpu.VMEM((CH, DA), jnp.float32),
        pltpu.VMEM((CH, DA), jnp.float32),
        pltpu.VMEM_SHARED((N, DA), jnp.float32),
        pltpu.SemaphoreType.DMA,
        pltpu.SemaphoreType.DMA,
        pltpu.SemaphoreType.DMA,
        pltpu.SemaphoreType.DMA,
        pltpu.SemaphoreType.DMA,
    ],
    compiler_params=pltpu.CompilerParams(use_tc_tiling_on_sc=False),
)(_sc_body)


def _tc_body(p_ref, x_ref, wm_ref, ws_ref, b_ref, o_ref):
  p = p_ref[0] + p_ref[1]
  deg = jnp.maximum(p[:, D:D + 1], 1.0)
  agg = p[:, :D] / deg
  h = jnp.dot(agg, wm_ref[...], preferred_element_type=jnp.float32)
  h = h + jnp.dot(x_ref[...], ws_ref[...], preferred_element_type=jnp.float32)
  h = h + b_ref[...]
  o_ref[...] = jnp.maximum(h, 0.0)


def kernel(x, edge_index, W_msg, W_self, b):
  xaug = jnp.zeros((N + 128, DA), jnp.float32)
  xaug = xaug.at[:N, :D].set(x).at[:N, D].set(1.0)
  src = edge_index[0].astype(jnp.int32)
  dst = edge_index[1].astype(jnp.int32)
  pad = E_PAD - E
  spread = jnp.arange(pad, dtype=jnp.int32) % 128
  enc = jnp.concatenate([src | (dst << 16), (N + spread) | (spread << 16)])
  enc = enc.reshape(NW, NCH, CH)
  dspread = jnp.arange(NW * 2 * CH, dtype=jnp.int32) % 128
  enc_dummy = ((N + dspread) | (dspread << 16)).reshape(NW, 2, CH)
  enc = jnp.concatenate([enc, enc_dummy], axis=1)
  enc = enc.reshape(NW * NIDX, CH)
  zeros_blk = jnp.zeros((CH, DA), jnp.float32)

  partial = _sc_gather_scatter(xaug, enc, zeros_blk)

  out = pl.pallas_call(
      _tc_body,
      grid=(N // BLK,),
      in_specs=[
          pl.BlockSpec((NC, BLK, DA), lambda i: (0, i, 0)),
          pl.BlockSpec((BLK, D), lambda i: (i, 0)),
          pl.BlockSpec((D, D), lambda i: (0, 0)),
          pl.BlockSpec((D, D), lambda i: (0, 0)),
          pl.BlockSpec((1, D), lambda i: (0, 0)),
      ],
      out_specs=pl.BlockSpec((BLK, D), lambda i: (i, 0)),
      out_shape=jax.ShapeDtypeStruct((N, D), jnp.float32),
  )(partial, x, W_msg, W_self, b.reshape(1, D))
  return out


# speedup vs baseline: 3.1878x; 1.2814x over previous
import functools

import jax
import jax.numpy as jnp
from jax import lax
from jax.experimental import pallas as pl
from jax.experimental.pallas import tpu as pltpu
from jax.experimental.pallas import tpu_sc as plsc

N = 10000
E = 320000
D = 128
DA = 144

NC = 2
NS = 16
NW = NC * NS

CH = 64
NCH = 162
E_PW = NCH * CH
E_PAD = NW * E_PW
NTRASH = 8
ROWS_PER_TILE = N // NS

BLK = 2000


def _sc_body(xaug_hbm, enc_hbm, zeros_hbm, out_hbm,
             pk_v, srcA, dstA, srcB, dstB, srcC, dstC, rowsA, rowsB, rowsC,
             acc, semi, gsA, gsB, gsC, ssA, ssB, ssC):
  c = lax.axis_index("c")
  s = lax.axis_index("s")
  wid = c * NS + s
  icp = pltpu.async_copy(enc_hbm.at[pl.ds(wid * E_PW, E_PW)], pk_v, semi)
  pltpu.sync_copy(zeros_hbm, rowsA)
  base = s * ROWS_PER_TILE
  nfull = ROWS_PER_TILE // CH
  rem = ROWS_PER_TILE - nfull * CH
  for k in range(nfull):
    pltpu.sync_copy(rowsA, acc.at[pl.ds(base + k * CH, CH)])
  pltpu.sync_copy(rowsA.at[pl.ds(0, rem)],
                  acc.at[pl.ds(base + nfull * CH, rem)])
  icp.wait()
  plsc.subcore_barrier()

  def unpack(j, src_b, dst_b):
    for k in range(CH // 16):
      v = pk_v[pl.ds(j * CH + k * 16, 16)]
      src_b[pl.ds(k * 16, 16)] = v & 0xFFFF
      dst_b[pl.ds(k * 16, 16)] = v >> 16

  unpack(0, srcA, dstA)
  pltpu.async_copy(xaug_hbm.at[srcA], rowsA, gsA)
  unpack(1, srcB, dstB)
  pltpu.async_copy(xaug_hbm.at[srcB], rowsB, gsB)
  unpack(2, srcC, dstC)
  pltpu.async_copy(xaug_hbm.at[srcC], rowsC, gsC)

  def body(jj, carry):
    j = jj * 3
    pltpu.make_async_copy(xaug_hbm.at[srcA], rowsA, gsA).wait()
    pltpu.async_copy(rowsA, acc.at[dstA], ssA, add=True)
    pltpu.make_async_copy(xaug_hbm.at[srcB], rowsB, gsB).wait()
    pltpu.async_copy(rowsB, acc.at[dstB], ssB, add=True)
    pltpu.make_async_copy(rowsA, acc.at[dstA], ssA).wait()
    unpack(jnp.minimum(j + 3, NCH - 1), srcA, dstA)
    pltpu.async_copy(xaug_hbm.at[srcA], rowsA, gsA)
    pltpu.make_async_copy(xaug_hbm.at[srcC], rowsC, gsC).wait()
    pltpu.async_copy(rowsC, acc.at[dstC], ssC, add=True)
    pltpu.make_async_copy(rowsB, acc.at[dstB], ssB).wait()
    unpack(jnp.minimum(j + 4, NCH - 1), srcB, dstB)
    pltpu.async_copy(xaug_hbm.at[srcB], rowsB, gsB)
    pltpu.make_async_copy(rowsC, acc.at[dstC], ssC).wait()
    unpack(jnp.minimum(j + 5, NCH - 1), srcC, dstC)
    pltpu.async_copy(xaug_hbm.at[srcC], rowsC, gsC)
    return carry

  lax.fori_loop(0, NCH // 3, body, 0)
  pltpu.make_async_copy(xaug_hbm.at[srcA], rowsA, gsA).wait()
  pltpu.make_async_copy(xaug_hbm.at[srcB], rowsB, gsB).wait()
  pltpu.make_async_copy(xaug_hbm.at[srcC], rowsC, gsC).wait()
  plsc.subcore_barrier()
  pltpu.sync_copy(acc.at[pl.ds(base, ROWS_PER_TILE)],
                  out_hbm.at[c, pl.ds(base, ROWS_PER_TILE)])


_sc_gather_scatter = functools.partial(
    pl.kernel,
    out_type=jax.ShapeDtypeStruct((NC, N, DA), jnp.float32),
    mesh=plsc.VectorSubcoreMesh(
        core_axis_name="c", subcore_axis_name="s", num_cores=NC,
        num_subcores=NS),
    scratch_types=[
        pltpu.VMEM((E_PW,), jnp.int32),
        pltpu.VMEM((CH,), jnp.int32),
        pltpu.VMEM((CH,), jnp.int32),
        pltpu.VMEM((CH,), jnp.int32),
        pltpu.VMEM((CH,), jnp.int32),
        pltpu.VMEM((CH,), jnp.int32),
        pltpu.VMEM((CH,), jnp.int32),
        pltpu.VMEM((CH, DA), jnp.float32),
        pltpu.VMEM((CH, DA), jnp.float32),
        pltpu.VMEM((CH, DA), jnp.float32),
        pltpu.VMEM_SHARED((N + NTRASH, DA), jnp.float32),
        pltpu.SemaphoreType.DMA,
        pltpu.SemaphoreType.DMA,
        pltpu.SemaphoreType.DMA,
        pltpu.SemaphoreType.DMA,
        pltpu.SemaphoreType.DMA,
        pltpu.SemaphoreType.DMA,
        pltpu.SemaphoreType.DMA,
    ],
    compiler_params=pltpu.CompilerParams(use_tc_tiling_on_sc=False),
)(_sc_body)


def _tc_body(p_ref, x_ref, wm_ref, ws_ref, b_ref, o_ref):
  p = p_ref[0] + p_ref[1]
  deg = jnp.maximum(p[:, D:D + 1], 1.0)
  agg = p[:, :D] / deg
  h = jnp.dot(agg, wm_ref[...], preferred_element_type=jnp.float32)
  h = h + jnp.dot(x_ref[...], ws_ref[...], preferred_element_type=jnp.float32)
  h = h + b_ref[...]
  o_ref[...] = jnp.maximum(h, 0.0)


def kernel(x, edge_index, W_msg, W_self, b):
  xaug = jnp.pad(x, ((0, 128), (0, DA - D)), constant_values=1.0)
  src = edge_index[0].astype(jnp.int32)
  dst = edge_index[1].astype(jnp.int32)
  pad = E_PAD - E
  spread = jnp.arange(pad, dtype=jnp.int32)
  pad_enc = (N + spread % 128) | ((N + spread % NTRASH) << 16)
  enc = jnp.concatenate([src | (dst << 16), pad_enc])
  zeros_blk = jnp.zeros((CH, DA), jnp.float32)

  partial = _sc_gather_scatter(xaug, enc, zeros_blk)

  out = pl.pallas_call(
      _tc_body,
      grid=(N // BLK,),
      in_specs=[
          pl.BlockSpec((NC, BLK, DA), lambda i: (0, i, 0)),
          pl.BlockSpec((BLK, D), lambda i: (i, 0)),
          pl.BlockSpec((D, D), lambda i: (0, 0)),
          pl.BlockSpec((D, D), lambda i: (0, 0)),
          pl.BlockSpec((1, D), lambda i: (0, 0)),
      ],
      out_specs=pl.BlockSpec((BLK, D), lambda i: (i, 0)),
      out_shape=jax.ShapeDtypeStruct((N, D), jnp.float32),
  )(partial, x, W_msg, W_self, b.reshape(1, D))
  return out


# speedup vs baseline: 3.2024x; 1.0046x over previous
import functools

import jax
import jax.numpy as jnp
from jax import lax
from jax.experimental import pallas as pl
from jax.experimental.pallas import tpu as pltpu
from jax.experimental.pallas import tpu_sc as plsc

N = 10000
E = 320000
D = 128
DA = 144

NC = 2
NS = 16
NW = NC * NS

CH = 64
NCH = 162
E_PW = NCH * CH
E_PAD = NW * E_PW
NTRASH = 8
ROWS_PER_TILE = N // NS

BLK = 2000


def _sc_body(xaug_hbm, enc_hbm, zeros_hbm, out_hbm,
             pk_v, srcA, dstA, srcB, dstB, srcC, dstC, rowsA, rowsB, rowsC,
             acc, semi, gsA, gsB, gsC, ssA, ssB, ssC):
  c = lax.axis_index("c")
  s = lax.axis_index("s")
  wid = c * NS + s
  icp = pltpu.async_copy(enc_hbm.at[pl.ds(wid * E_PW, E_PW)], pk_v, semi)
  pltpu.sync_copy(zeros_hbm, rowsA)
  base = s * ROWS_PER_TILE
  nfull = ROWS_PER_TILE // CH
  rem = ROWS_PER_TILE - nfull * CH
  for k in range(nfull):
    pltpu.sync_copy(rowsA, acc.at[pl.ds(base + k * CH, CH)])
  pltpu.sync_copy(rowsA.at[pl.ds(0, rem)],
                  acc.at[pl.ds(base + nfull * CH, rem)])
  icp.wait()
  plsc.subcore_barrier()

  def unpack(j, src_b, dst_b):
    for k in range(CH // 16):
      v = pk_v[pl.ds(j * CH + k * 16, 16)]
      src_b[pl.ds(k * 16, 16)] = v & 0xFFFF
      dst_b[pl.ds(k * 16, 16)] = v >> 16

  unpack(0, srcA, dstA)
  pltpu.async_copy(xaug_hbm.at[srcA], rowsA, gsA)
  unpack(1, srcB, dstB)
  pltpu.async_copy(xaug_hbm.at[srcB], rowsB, gsB)
  unpack(2, srcC, dstC)
  pltpu.async_copy(xaug_hbm.at[srcC], rowsC, gsC)

  def body(jj, carry):
    j = jj * 3
    pltpu.make_async_copy(xaug_hbm.at[srcA], rowsA, gsA).wait()
    pltpu.async_copy(rowsA, acc.at[dstA], ssA, add=True)
    pltpu.make_async_copy(xaug_hbm.at[srcB], rowsB, gsB).wait()
    pltpu.async_copy(rowsB, acc.at[dstB], ssB, add=True)
    pltpu.make_async_copy(rowsA, acc.at[dstA], ssA).wait()
    unpack(jnp.minimum(j + 3, NCH - 1), srcA, dstA)
    pltpu.async_copy(xaug_hbm.at[srcA], rowsA, gsA)
    pltpu.make_async_copy(xaug_hbm.at[srcC], rowsC, gsC).wait()
    pltpu.async_copy(rowsC, acc.at[dstC], ssC, add=True)
    pltpu.make_async_copy(rowsB, acc.at[dstB], ssB).wait()
    unpack(jnp.minimum(j + 4, NCH - 1), srcB, dstB)
    pltpu.async_copy(xaug_hbm.at[srcB], rowsB, gsB)
    pltpu.make_async_copy(rowsC, acc.at[dstC], ssC).wait()
    unpack(jnp.minimum(j + 5, NCH - 1), srcC, dstC)
    pltpu.async_copy(xaug_hbm.at[srcC], rowsC, gsC)
    return carry

  lax.fori_loop(0, NCH // 3, body, 0)
  pltpu.make_async_copy(xaug_hbm.at[srcA], rowsA, gsA).wait()
  pltpu.make_async_copy(xaug_hbm.at[srcB], rowsB, gsB).wait()
  pltpu.make_async_copy(xaug_hbm.at[srcC], rowsC, gsC).wait()
  plsc.subcore_barrier()
  pltpu.sync_copy(acc.at[pl.ds(base, ROWS_PER_TILE)],
                  out_hbm.at[c, pl.ds(base, ROWS_PER_TILE)])


_sc_gather_scatter = functools.partial(
    pl.kernel,
    out_type=jax.ShapeDtypeStruct((NC, N, DA), jnp.float32),
    mesh=plsc.VectorSubcoreMesh(
        core_axis_name="c", subcore_axis_name="s", num_cores=NC,
        num_subcores=NS),
    scratch_types=[
        pltpu.VMEM((E_PW,), jnp.int32),
        pltpu.VMEM((CH,), jnp.int32),
        pltpu.VMEM((CH,), jnp.int32),
        pltpu.VMEM((CH,), jnp.int32),
        pltpu.VMEM((CH,), jnp.int32),
        pltpu.VMEM((CH,), jnp.int32),
        pltpu.VMEM((CH,), jnp.int32),
        pltpu.VMEM((CH, DA), jnp.float32),
        pltpu.VMEM((CH, DA), jnp.float32),
        pltpu.VMEM((CH, DA), jnp.float32),
        pltpu.VMEM_SHARED((N + NTRASH, DA), jnp.float32),
        pltpu.SemaphoreType.DMA,
        pltpu.SemaphoreType.DMA,
        pltpu.SemaphoreType.DMA,
        pltpu.SemaphoreType.DMA,
        pltpu.SemaphoreType.DMA,
        pltpu.SemaphoreType.DMA,
        pltpu.SemaphoreType.DMA,
    ],
    compiler_params=pltpu.CompilerParams(use_tc_tiling_on_sc=False),
)(_sc_body)


def _tc_body(p_ref, x_ref, wm_ref, ws_ref, b_ref, o_ref):
  p = p_ref[0] + p_ref[1]
  deg = jnp.maximum(p[:, D:D + 1], 1.0)
  agg = p[:, :D] / deg
  h = jnp.dot(agg, wm_ref[...], preferred_element_type=jnp.float32)
  h = h + jnp.dot(x_ref[...], ws_ref[...], preferred_element_type=jnp.float32)
  h = h + b_ref[...]
  o_ref[...] = jnp.maximum(h, 0.0)


def kernel(x, edge_index, W_msg, W_self, b):
  xaug = jnp.pad(x, ((0, 128), (0, DA - D)), constant_values=1.0)
  pad = E_PAD - E
  mult = jnp.array([[1], [1 << 16]], dtype=jnp.int32)
  enc_edges = jnp.sum(edge_index.astype(jnp.int32) * mult, axis=0,
                      dtype=jnp.int32)
  spread = jnp.arange(pad, dtype=jnp.int32)
  pad_enc = (N + spread % 128) | ((N + spread % NTRASH) << 16)
  enc = jnp.concatenate([enc_edges, pad_enc])
  zeros_blk = jnp.zeros((CH, DA), jnp.float32)

  partial = _sc_gather_scatter(xaug, enc, zeros_blk)

  out = pl.pallas_call(
      _tc_body,
      grid=(N // BLK,),
      in_specs=[
          pl.BlockSpec((NC, BLK, DA), lambda i: (0, i, 0)),
          pl.BlockSpec((BLK, D), lambda i: (i, 0)),
          pl.BlockSpec((D, D), lambda i: (0, 0)),
          pl.BlockSpec((D, D), lambda i: (0, 0)),
          pl.BlockSpec((1, D), lambda i: (0, 0)),
      ],
      out_specs=pl.BlockSpec((BLK, D), lambda i: (i, 0)),
      out_shape=jax.ShapeDtypeStruct((N, D), jnp.float32),
  )(partial, x, W_msg, W_self, b.reshape(1, D))
  return out


# speedup vs baseline: 4.4222x; 1.3809x over previous
import functools

import jax
import jax.numpy as jnp
from jax import lax
from jax.experimental import pallas as pl
from jax.experimental.pallas import tpu as pltpu
from jax.experimental.pallas import tpu_sc as plsc

N = 10000
E = 320000
D = 128
DG = 16

NC = 2
NS = 16
NW = NC * NS

CH = 64
NCH = 162
E_PW = NCH * CH
E_PAD = NW * E_PW
NTRASH = 8
ROWS_PER_TILE = N // NS
ND = N + 240


def _sc_body(x_hbm, enc_hbm, zeros_hbm, feat_hbm, deg_hbm,
             pk_v, srcA, dstA, srcB, dstB, srcC, dstC, rowsA, rowsB, rowsC,
             ones_v, zer16_v, facc, dacc,
             semi, gsA, gsB, gsC, ssA, ssB, ssC, dsA, dsB, dsC):
  c = lax.axis_index("c")
  s = lax.axis_index("s")
  wid = c * NS + s
  icp = pltpu.async_copy(enc_hbm.at[pl.ds(wid * E_PW, E_PW)], pk_v, semi)
  for r in range(CH):
    ones_v[r, :] = jnp.ones((DG,), jnp.float32)
    zer16_v[r, :] = jnp.zeros((DG,), jnp.float32)
  pltpu.sync_copy(zeros_hbm, rowsA)
  base = s * ROWS_PER_TILE
  nfull = ROWS_PER_TILE // CH
  rem = ROWS_PER_TILE - nfull * CH
  for k in range(nfull):
    pltpu.sync_copy(rowsA, facc.at[pl.ds(base + k * CH, CH)])
    pltpu.sync_copy(zer16_v, dacc.at[pl.ds(base + k * CH, CH)])
  pltpu.sync_copy(rowsA.at[pl.ds(0, rem)],
                  facc.at[pl.ds(base + nfull * CH, rem)])
  pltpu.sync_copy(zer16_v.at[pl.ds(0, rem)],
                  dacc.at[pl.ds(base + nfull * CH, rem)])
  icp.wait()
  plsc.subcore_barrier()

  def unpack(j, src_b, dst_b):
    for k in range(CH // 16):
      v = pk_v[pl.ds(j * CH + k * 16, 16)]
      src_b[pl.ds(k * 16, 16)] = v & 0xFFFF
      dst_b[pl.ds(k * 16, 16)] = v >> 16

  unpack(0, srcA, dstA)
  pltpu.async_copy(x_hbm.at[srcA], rowsA, gsA)
  unpack(1, srcB, dstB)
  pltpu.async_copy(x_hbm.at[srcB], rowsB, gsB)
  unpack(2, srcC, dstC)
  pltpu.async_copy(x_hbm.at[srcC], rowsC, gsC)

  def step(rows, src_b, dst_b, gs, ss, ds, jnext):
    pltpu.make_async_copy(x_hbm.at[src_b], rows, gs).wait()
    pltpu.async_copy(rows, facc.at[dst_b], ss, add=True)
    pltpu.async_copy(ones_v, dacc.at[dst_b], ds, add=True)
    return None

  def refill(rows, src_b, dst_b, gs, ss, ds, jnext):
    pltpu.make_async_copy(rows, facc.at[dst_b], ss).wait()
    pltpu.make_async_copy(ones_v, dacc.at[dst_b], ds).wait()
    unpack(jnext, src_b, dst_b)
    pltpu.async_copy(x_hbm.at[src_b], rows, gs)
    return None

  def body(jj, carry):
    j = jj * 3
    step(rowsA, srcA, dstA, gsA, ssA, dsA, j)
    step(rowsB, srcB, dstB, gsB, ssB, dsB, j + 1)
    refill(rowsA, srcA, dstA, gsA, ssA, dsA, jnp.minimum(j + 3, NCH - 1))
    step(rowsC, srcC, dstC, gsC, ssC, dsC, j + 2)
    refill(rowsB, srcB, dstB, gsB, ssB, dsB, jnp.minimum(j + 4, NCH - 1))
    refill(rowsC, srcC, dstC, gsC, ssC, dsC, jnp.minimum(j + 5, NCH - 1))
    return carry

  lax.fori_loop(0, NCH // 3, body, 0)
  pltpu.make_async_copy(x_hbm.at[srcA], rowsA, gsA).wait()
  pltpu.make_async_copy(x_hbm.at[srcB], rowsB, gsB).wait()
  pltpu.make_async_copy(x_hbm.at[srcC], rowsC, gsC).wait()
  plsc.subcore_barrier()
  pltpu.sync_copy(facc.at[pl.ds(base, ROWS_PER_TILE)],
                  feat_hbm.at[c, pl.ds(base, ROWS_PER_TILE)])
  pltpu.sync_copy(dacc.at[pl.ds(base, ROWS_PER_TILE)],
                  deg_hbm.at[c, pl.ds(base, ROWS_PER_TILE)])


_sc_gather_scatter = functools.partial(
    pl.kernel,
    out_type=(jax.ShapeDtypeStruct((NC, N, D), jnp.float32),
              jax.ShapeDtypeStruct((NC, ND, DG), jnp.float32)),
    mesh=plsc.VectorSubcoreMesh(
        core_axis_name="c", subcore_axis_name="s", num_cores=NC,
        num_subcores=NS),
    scratch_types=[
        pltpu.VMEM((E_PW,), jnp.int32),
        pltpu.VMEM((CH,), jnp.int32),
        pltpu.VMEM((CH,), jnp.int32),
        pltpu.VMEM((CH,), jnp.int32),
        pltpu.VMEM((CH,), jnp.int32),
        pltpu.VMEM((CH,), jnp.int32),
        pltpu.VMEM((CH,), jnp.int32),
        pltpu.VMEM((CH, D), jnp.float32),
        pltpu.VMEM((CH, D), jnp.float32),
        pltpu.VMEM((CH, D), jnp.float32),
        pltpu.VMEM((CH, DG), jnp.float32),
        pltpu.VMEM((CH, DG), jnp.float32),
        pltpu.VMEM_SHARED((N + NTRASH, D), jnp.float32),
        pltpu.VMEM_SHARED((ND, DG), jnp.float32),
        pltpu.SemaphoreType.DMA,
        pltpu.SemaphoreType.DMA,
        pltpu.SemaphoreType.DMA,
        pltpu.SemaphoreType.DMA,
        pltpu.SemaphoreType.DMA,
        pltpu.SemaphoreType.DMA,
        pltpu.SemaphoreType.DMA,
        pltpu.SemaphoreType.DMA,
        pltpu.SemaphoreType.DMA,
        pltpu.SemaphoreType.DMA,
    ],
    compiler_params=pltpu.CompilerParams(use_tc_tiling_on_sc=False),
)(_sc_body)

NDROW = ND * DG // 128


def _tc_body(pf_ref, df_ref, x_ref, wm_ref, ws_ref, b_ref, o_ref):
  lanes = lax.broadcasted_iota(jnp.int32, (128, 8), 0)
  picks = lax.broadcasted_iota(jnp.int32, (128, 8), 1) * DG
  sel = (lanes == picks).astype(jnp.float32)
  dd = df_ref[0] + df_ref[1]
  deg8 = jnp.dot(dd, sel, preferred_element_type=jnp.float32)
  rdeg = 1.0 / jnp.maximum(deg8, 1.0)
  rdeg = jnp.broadcast_to(rdeg[:, :, None], (NDROW, 8, 128))
  rdeg = rdeg.reshape(NDROW * 8, 128)[:N]
  agg = (pf_ref[0] + pf_ref[1]) * rdeg
  h = jnp.dot(agg, wm_ref[...], preferred_element_type=jnp.float32)
  h = h + jnp.dot(x_ref[...], ws_ref[...], preferred_element_type=jnp.float32)
  h = h + b_ref[...]
  o_ref[...] = jnp.maximum(h, 0.0)


def kernel(x, edge_index, W_msg, W_self, b):
  pad = E_PAD - E
  mult = jnp.array([[1], [1 << 16]], dtype=jnp.int32)
  enc_edges = jnp.sum(edge_index.astype(jnp.int32) * mult, axis=0,
                      dtype=jnp.int32)
  spread = jnp.arange(pad, dtype=jnp.int32)
  pad_enc = (spread % 128) | ((N + spread % NTRASH) << 16)
  enc = jnp.concatenate([enc_edges, pad_enc])
  zeros_blk = jnp.zeros((CH, D), jnp.float32)

  feat, deg = _sc_gather_scatter(x, enc, zeros_blk)
  degr = deg.reshape(NC, NDROW, 128)

  out = pl.pallas_call(
      _tc_body,
      grid=(1,),
      in_specs=[
          pl.BlockSpec((NC, N, D), lambda i: (0, 0, 0)),
          pl.BlockSpec((NC, NDROW, 128), lambda i: (0, 0, 0)),
          pl.BlockSpec((N, D), lambda i: (0, 0)),
          pl.BlockSpec((D, D), lambda i: (0, 0)),
          pl.BlockSpec((D, D), lambda i: (0, 0)),
          pl.BlockSpec((1, D), lambda i: (0, 0)),
      ],
      out_specs=pl.BlockSpec((N, D), lambda i: (0, 0)),
      out_shape=jax.ShapeDtypeStruct((N, D), jnp.float32),
  )(feat, degr, x, W_msg, W_self, b.reshape(1, D))
  return out


# speedup vs baseline: 4.7649x; 1.0775x over previous
import functools

import jax
import jax.numpy as jnp
from jax import lax
from jax.experimental import pallas as pl
from jax.experimental.pallas import tpu as pltpu
from jax.experimental.pallas import tpu_sc as plsc

N = 10000
E = 320000
D = 128
DG = 16

NC = 2
NS = 16
NW = NC * NS

CH = 96
NCH = 108
E_PW = NCH * CH
E_PAD = NW * E_PW
NTRASH = 8
ROWS_PER_TILE = N // NS
ND = N + 240


def _sc_body(x_hbm, ei_hbm, zeros_hbm, feat_hbm, deg_hbm,
             idxA, idxB, idxC, keepA, keepB, keepC,
             rowsA, rowsB, rowsC, ones_v, zer16_v,
             facc, dacc,
             diA, diB, diC, gsA, gsB, gsC, ssA, ssB, ssC, dsA, dsB, dsC):
  c = lax.axis_index("c")
  s = lax.axis_index("s")
  wid = c * NS + s
  ebase = wid * E_PW

  def idx_dma(j, idx_b, di):
    pltpu.async_copy(ei_hbm.at[0, pl.ds(ebase + j * CH, CH)],
                     idx_b.at[0], di)
    pltpu.async_copy(ei_hbm.at[1, pl.ds(ebase + j * CH, CH)],
                     idx_b.at[1], di)

  def idx_wait(idx_b, di):
    pltpu.make_async_copy(ei_hbm.at[0, pl.ds(0, CH)], idx_b.at[0], di).wait()
    pltpu.make_async_copy(ei_hbm.at[1, pl.ds(0, CH)], idx_b.at[1], di).wait()

  idx_dma(0, idxA, diA)
  idx_dma(1, idxB, diB)
  idx_dma(2, idxC, diC)
  for r in range(CH):
    ones_v[r, :] = jnp.ones((DG,), jnp.float32)
    zer16_v[r, :] = jnp.zeros((DG,), jnp.float32)
  pltpu.sync_copy(zeros_hbm, rowsA)
  base = s * ROWS_PER_TILE
  nfull = ROWS_PER_TILE // CH
  rem = ROWS_PER_TILE - nfull * CH
  for k in range(nfull):
    pltpu.sync_copy(rowsA, facc.at[pl.ds(base + k * CH, CH)])
    pltpu.sync_copy(zer16_v, dacc.at[pl.ds(base + k * CH, CH)])
  pltpu.sync_copy(rowsA.at[pl.ds(0, rem)],
                  facc.at[pl.ds(base + nfull * CH, rem)])
  pltpu.sync_copy(zer16_v.at[pl.ds(0, rem)],
                  dacc.at[pl.ds(base + nfull * CH, rem)])
  plsc.subcore_barrier()

  idx_wait(idxA, diA)
  pltpu.async_copy(x_hbm.at[idxA.at[0]], rowsA, gsA)
  idx_wait(idxB, diB)
  pltpu.async_copy(x_hbm.at[idxB.at[0]], rowsB, gsB)
  idx_wait(idxC, diC)
  pltpu.async_copy(x_hbm.at[idxC.at[0]], rowsC, gsC)

  def step(rows, idx_b, keep, gs, ss, ds, di, jpre):
    pltpu.make_async_copy(x_hbm.at[idx_b.at[0]], rows, gs).wait()
    for k in range(CH // 16):
      keep[pl.ds(k * 16, 16)] = idx_b[1, pl.ds(k * 16, 16)]
    idx_dma(jpre, idx_b, di)
    pltpu.async_copy(rows, facc.at[keep], ss, add=True)
    pltpu.async_copy(ones_v, dacc.at[keep], ds, add=True)

  def refill(rows, idx_b, keep, gs, ss, ds, di):
    pltpu.make_async_copy(rows, facc.at[keep], ss).wait()
    pltpu.make_async_copy(ones_v, dacc.at[keep], ds).wait()
    idx_wait(idx_b, di)
    pltpu.async_copy(x_hbm.at[idx_b.at[0]], rows, gs)

  def body(jj, carry):
    j = jj * 3
    step(rowsA, idxA, keepA, gsA, ssA, dsA, diA, jnp.minimum(j + 3, NCH - 1))
    step(rowsB, idxB, keepB, gsB, ssB, dsB, diB, jnp.minimum(j + 4, NCH - 1))
    refill(rowsA, idxA, keepA, gsA, ssA, dsA, diA)
    step(rowsC, idxC, keepC, gsC, ssC, dsC, diC, jnp.minimum(j + 5, NCH - 1))
    refill(rowsB, idxB, keepB, gsB, ssB, dsB, diB)
    refill(rowsC, idxC, keepC, gsC, ssC, dsC, diC)
    return carry

  lax.fori_loop(0, NCH // 3, body, 0)
  pltpu.make_async_copy(x_hbm.at[idxA.at[0]], rowsA, gsA).wait()
  pltpu.make_async_copy(x_hbm.at[idxB.at[0]], rowsB, gsB).wait()
  pltpu.make_async_copy(x_hbm.at[idxC.at[0]], rowsC, gsC).wait()
  plsc.subcore_barrier()
  pltpu.sync_copy(facc.at[pl.ds(base, ROWS_PER_TILE)],
                  feat_hbm.at[c, pl.ds(base, ROWS_PER_TILE)])
  pltpu.sync_copy(dacc.at[pl.ds(base, ROWS_PER_TILE)],
                  deg_hbm.at[c, pl.ds(base, ROWS_PER_TILE)])


_sc_gather_scatter = functools.partial(
    pl.kernel,
    out_type=(jax.ShapeDtypeStruct((NC, N, D), jnp.float32),
              jax.ShapeDtypeStruct((NC, ND, DG), jnp.float32)),
    mesh=plsc.VectorSubcoreMesh(
        core_axis_name="c", subcore_axis_name="s", num_cores=NC,
        num_subcores=NS),
    scratch_types=[
        pltpu.VMEM((2, CH), jnp.int32),
        pltpu.VMEM((2, CH), jnp.int32),
        pltpu.VMEM((2, CH), jnp.int32),
        pltpu.VMEM((CH,), jnp.int32),
        pltpu.VMEM((CH,), jnp.int32),
        pltpu.VMEM((CH,), jnp.int32),
        pltpu.VMEM((CH, D), jnp.float32),
        pltpu.VMEM((CH, D), jnp.float32),
        pltpu.VMEM((CH, D), jnp.float32),
        pltpu.VMEM((CH, DG), jnp.float32),
        pltpu.VMEM((CH, DG), jnp.float32),
        pltpu.VMEM_SHARED((N + NTRASH, D), jnp.float32),
        pltpu.VMEM_SHARED((N + NTRASH, DG), jnp.float32),
        pltpu.SemaphoreType.DMA,
        pltpu.SemaphoreType.DMA,
        pltpu.SemaphoreType.DMA,
        pltpu.SemaphoreType.DMA,
        pltpu.SemaphoreType.DMA,
        pltpu.SemaphoreType.DMA,
        pltpu.SemaphoreType.DMA,
        pltpu.SemaphoreType.DMA,
        pltpu.SemaphoreType.DMA,
        pltpu.SemaphoreType.DMA,
        pltpu.SemaphoreType.DMA,
        pltpu.SemaphoreType.DMA,
    ],
    compiler_params=pltpu.CompilerParams(use_tc_tiling_on_sc=False),
)(_sc_body)

NDROW = ND * DG // 128


def _tc_body(pf_ref, df_ref, x_ref, wm_ref, ws_ref, b_ref, o_ref):
  lanes = lax.broadcasted_iota(jnp.int32, (128, 8), 0)
  picks = lax.broadcasted_iota(jnp.int32, (128, 8), 1) * DG
  sel = (lanes == picks).astype(jnp.float32)
  dd = df_ref[0] + df_ref[1]
  deg8 = jnp.dot(dd, sel, preferred_element_type=jnp.float32)
  rdeg = 1.0 / jnp.maximum(deg8, 1.0)
  rdeg = jnp.broadcast_to(rdeg[:, :, None], (NDROW, 8, 128))
  rdeg = rdeg.reshape(NDROW * 8, 128)[:N]
  agg = (pf_ref[0] + pf_ref[1]) * rdeg
  h = jnp.dot(agg, wm_ref[...], preferred_element_type=jnp.float32)
  h = h + jnp.dot(x_ref[...], ws_ref[...], preferred_element_type=jnp.float32)
  h = h + b_ref[...]
  o_ref[...] = jnp.maximum(h, 0.0)


def kernel(x, edge_index, W_msg, W_self, b):
  pad = E_PAD - E
  spread = jnp.arange(pad, dtype=jnp.int32)
  pad_pairs = jnp.stack([spread % 128, N + spread % NTRASH])
  ei = jnp.concatenate([edge_index.astype(jnp.int32), pad_pairs], axis=1)
  zeros_blk = jnp.zeros((CH, D), jnp.float32)

  feat, deg = _sc_gather_scatter(x, ei, zeros_blk)
  degr = deg.reshape(NC, NDROW, 128)

  out = pl.pallas_call(
      _tc_body,
      grid=(1,),
      in_specs=[
          pl.BlockSpec((NC, N, D), lambda i: (0, 0, 0)),
          pl.BlockSpec((NC, NDROW, 128), lambda i: (0, 0, 0)),
          pl.BlockSpec((N, D), lambda i: (0, 0)),
          pl.BlockSpec((D, D), lambda i: (0, 0)),
          pl.BlockSpec((D, D), lambda i: (0, 0)),
          pl.BlockSpec((1, D), lambda i: (0, 0)),
      ],
      out_specs=pl.BlockSpec((N, D), lambda i: (0, 0)),
      out_shape=jax.ShapeDtypeStruct((N, D), jnp.float32),
  )(feat, degr, x, W_msg, W_self, b.reshape(1, D))
  return out
